# baseline (speedup 1.0000x reference)
import functools

import jax
import jax.numpy as jnp
from jax import lax
from jax.experimental import pallas as pl
from jax.experimental.pallas import tpu as pltpu
from jax.experimental.pallas import tpu_sc as plsc

N = 10000
NP = N + 8
D = 128
E = 320000
NC = 2
NS = 16
NW = NC * NS
CHUNK = 128
EPW = 10240
CPW = EPW // CHUNK
GB = 16
NG = CPW // GB
RCH = 80
NRCH = N // RCH
QCH = 400
NQ = N // QCH

_MESH = plsc.VectorSubcoreMesh(core_axis_name="c", subcore_axis_name="s")


def _qrsqrt(x):
    magic = jnp.full((16,), 0x5F3759DF, jnp.int32)
    one = jnp.full((16,), 1, jnp.int32)
    half = jnp.full((16,), 0.5, jnp.float32)
    th = jnp.full((16,), 1.5, jnp.float32)
    i = lax.bitcast_convert_type(x, jnp.int32)
    y = lax.bitcast_convert_type(
        magic - lax.shift_right_arithmetic(i, one), jnp.float32)
    hx = half * x
    for _ in range(3):
        y = y * (th - hx * y * y)
    return y


@functools.partial(
    pl.kernel,
    out_type=[
        jax.ShapeDtypeStruct((NP, D), jnp.float32),
        jax.ShapeDtypeStruct((N,), jnp.float32),
    ],
    mesh=_MESH,
    scratch_types=[
        pltpu.VMEM((GB, CHUNK), jnp.int32),
        pltpu.VMEM((128,), jnp.float32),
        pltpu.VMEM((1024,), jnp.float32),
        pltpu.VMEM((QCH,), jnp.float32),
        pltpu.VMEM((RCH, D), jnp.float32),
        pltpu.VMEM((RCH,), jnp.float32),
        pltpu.VMEM_SHARED((NP,), jnp.float32),
        pltpu.SemaphoreType.DMA,
    ],
)
def _deg_kernel(idx_hbm, feat_hbm, fsrc_hbm, nr_hbm, idx_v, ones_v, zero_v,
                q_v, frow_v, nrow_v, hist, sem):
    cid = lax.axis_index("c")
    sid = lax.axis_index("s")

    for i in range(8):
        ones_v[pl.ds(i * 16, 16)] = jnp.ones((16,), jnp.float32)
    for i in range(64):
        zero_v[pl.ds(i * 16, 16)] = jnp.zeros((16,), jnp.float32)

    @pl.when(sid < 10)
    def _():
        pltpu.sync_copy(zero_v.at[pl.ds(0, 1000)],
                        hist.at[pl.ds(sid * 1000, 1000)])

    plsc.subcore_barrier()

    def blk(b, carry):
        wid = sid * 2 + b

        def group(g, carry2):
            pltpu.sync_copy(idx_hbm.at[cid, wid, pl.ds(g * GB, GB)], idx_v)

            def body(j, carry3):
                pltpu.sync_copy(ones_v, hist.at[idx_v.at[j]], add=True)
                return carry3

            lax.fori_loop(0, GB, body, 0)
            return carry2

        lax.fori_loop(0, NG, group, 0)
        return carry

    lax.fori_loop(0, 2, blk, 0)

    plsc.subcore_barrier()

    def rs(m, carry):
        k = sid + m * NS

        @pl.when(k < NQ)
        def _():
            pltpu.sync_copy(hist.at[pl.ds(k * QCH, QCH)], q_v)
            for t in range(QCH // 16):
                x = jnp.maximum(q_v[pl.ds(t * 16, 16)], 1.0)
                q_v[pl.ds(t * 16, 16)] = _qrsqrt(x)

            @pl.when(cid == 0)
            def _():
                pltpu.sync_copy(q_v, hist.at[pl.ds(k * QCH, QCH)])

            @pl.when(cid == 1)
            def _():
                pltpu.sync_copy(q_v, nr_hbm.at[pl.ds(k * QCH, QCH)])

        return carry

    lax.fori_loop(0, (NQ + NS - 1) // NS, rs, 0)

    plsc.subcore_barrier()

    @pl.when(cid == 0)
    def _():
        def sc(m, carry):
            k = sid + m * NS

            @pl.when(k < NRCH)
            def _():
                pltpu.sync_copy(feat_hbm.at[pl.ds(k * RCH, RCH)], frow_v)
                pltpu.sync_copy(hist.at[pl.ds(k * RCH, RCH)], nrow_v)

                def rowblk(rb, carry2):
                    nv = nrow_v[pl.ds(rb * 16, 16)]
                    for rr in range(16):
                        r = rb * 16 + rr
                        v = nv[rr]
                        for c in range(D // 16):
                            frow_v[r, pl.ds(c * 16, 16)] = (
                                frow_v[r, pl.ds(c * 16, 16)] * v)
                    return carry2

                lax.fori_loop(0, RCH // 16, rowblk, 0)
                pltpu.sync_copy(frow_v, fsrc_hbm.at[pl.ds(k * RCH, RCH)])

            return carry

        lax.fori_loop(0, (NRCH + NS - 1) // NS, sc, 0)


@functools.partial(
    pl.kernel,
    out_type=jax.ShapeDtypeStruct((NC, N, D), jnp.float32),
    mesh=_MESH,
    scratch_types=[
        pltpu.VMEM((GB, CHUNK), jnp.int32),
        pltpu.VMEM((GB, CHUNK), jnp.int32),
        pltpu.VMEM((CHUNK, D), jnp.float32),
        pltpu.VMEM((CHUNK, D), jnp.float32),
        pltpu.VMEM((RCH, D), jnp.float32),
        pltpu.VMEM((RCH,), jnp.float32),
        pltpu.VMEM_SHARED((NP, D), jnp.float32),
        pltpu.SemaphoreType.DMA,
        pltpu.SemaphoreType.DMA,
    ],
)
def _spmm_kernel(idx_hbm, feat_hbm, nr_hbm, out_hbm, src_v, dst_v, rows_a,
                 rows_b, zrow_v, nrow_v, accum, sem_a, sem_b):
    cid = lax.axis_index("c")
    sid = lax.axis_index("s")
    wid = cid * NS + sid

    def zfill(j, carry):
        for c in range(D // 16):
            zrow_v[j, pl.ds(c * 16, 16)] = jnp.zeros((16,), jnp.float32)
        return carry

    lax.fori_loop(0, RCH, zfill, 0)

    def zacc(m, carry):
        k = sid + m * NS

        @pl.when(k < NRCH)
        def _():
            pltpu.sync_copy(zrow_v, accum.at[pl.ds(k * RCH, RCH)])

        return carry

    lax.fori_loop(0, (NRCH + NS - 1) // NS, zacc, 0)
    plsc.subcore_barrier()

    def group(g, carry):
        pltpu.sync_copy(idx_hbm.at[0, wid, pl.ds(g * GB, GB)], src_v)
        pltpu.sync_copy(idx_hbm.at[1, wid, pl.ds(g * GB, GB)], dst_v)

        pltpu.async_copy(feat_hbm.at[src_v.at[0]], rows_a, sem_a)

        def pair(p, carry2):
            j0 = 2 * p
            pltpu.async_copy(feat_hbm.at[src_v.at[j0 + 1]], rows_b, sem_b)
            pltpu.make_async_copy(feat_hbm.at[src_v.at[j0]], rows_a,
                                  sem_a).wait()
            pltpu.sync_copy(rows_a, accum.at[dst_v.at[j0]], add=True)

            @pl.when(j0 + 2 < GB)
            def _():
                pltpu.async_copy(feat_hbm.at[src_v.at[j0 + 2]], rows_a, sem_a)

            pltpu.make_async_copy(feat_hbm.at[src_v.at[j0 + 1]], rows_b,
                                  sem_b).wait()
            pltpu.sync_copy(rows_b, accum.at[dst_v.at[j0 + 1]], add=True)
            return carry2

        lax.fori_loop(0, GB // 2, pair, 0)
        return carry

    lax.fori_loop(0, NG, group, 0)

    plsc.subcore_barrier()

    def copyout(m, carry):
        k = sid + m * NS

        @pl.when(k < NRCH)
        def _():
            pltpu.sync_copy(accum.at[pl.ds(k * RCH, RCH)], zrow_v)
            pltpu.sync_copy(nr_hbm.at[pl.ds(k * RCH, RCH)], nrow_v)

            def rowblk(rb, carry2):
                nv = nrow_v[pl.ds(rb * 16, 16)]
                for rr in range(16):
                    r = rb * 16 + rr
                    v = nv[rr]
                    for c in range(D // 16):
                        zrow_v[r, pl.ds(c * 16, 16)] = (
                            zrow_v[r, pl.ds(c * 16, 16)] * v)
                return carry2

            lax.fori_loop(0, RCH // 16, rowblk, 0)
            pltpu.sync_copy(zrow_v, out_hbm.at[cid, pl.ds(k * RCH, RCH)])

        return carry

    lax.fori_loop(0, (NRCH + NS - 1) // NS, copyout, 0)


RB = 1000


def _out_body(accp_ref, w_ref, out_ref):
    acc = accp_ref[0] + accp_ref[1]
    out_ref[...] = jnp.dot(acc, w_ref[...],
                           preferred_element_type=jnp.float32)


_combine = pl.pallas_call(
    _out_body,
    grid=(N // RB,),
    in_specs=[
        pl.BlockSpec((NC, RB, D), lambda i: (0, i, 0)),
        pl.BlockSpec((D, D), lambda i: (0, 0)),
    ],
    out_specs=pl.BlockSpec((RB, D), lambda i: (i, 0)),
    out_shape=jax.ShapeDtypeStruct((N, D), jnp.float32),
)


@jax.jit
def kernel(feat, edge_index, weight):
    ei = edge_index.astype(jnp.int32).reshape(2, NW, E // NW)
    idx = jnp.pad(ei, ((0, 0), (0, 0), (0, EPW - E // NW)),
                  constant_values=N).reshape(2, NW, CPW, CHUNK)
    feat_src, norm_r = _deg_kernel(idx, feat)
    accp = _spmm_kernel(idx, feat_src, norm_r)
    return _combine(accp, weight)


# speedup vs baseline: 2.4185x; 2.4185x over previous
import functools

import jax
import jax.numpy as jnp
from jax import lax
from jax.experimental import pallas as pl
from jax.experimental.pallas import tpu as pltpu
from jax.experimental.pallas import tpu_sc as plsc

N = 10000
NP = 10240
D = 128
E = 320000
NC = 2
NS = 16
NW = NC * NS
CHUNK = 128
EPW = 10240
CPW = EPW // CHUNK
GB = 16
NG = CPW // GB
RCH = 80
NRCH = N // RCH
QCH = 400
NQ = N // QCH

_MESH = plsc.VectorSubcoreMesh(core_axis_name="c", subcore_axis_name="s")


def _qrsqrt(x):
    magic = jnp.full((16,), 0x5F3759DF, jnp.int32)
    one = jnp.full((16,), 1, jnp.int32)
    half = jnp.full((16,), 0.5, jnp.float32)
    th = jnp.full((16,), 1.5, jnp.float32)
    i = lax.bitcast_convert_type(x, jnp.int32)
    y = lax.bitcast_convert_type(
        magic - lax.shift_right_arithmetic(i, one), jnp.float32)
    hx = half * x
    for _ in range(3):
        y = y * (th - hx * y * y)
    return y


@functools.partial(
    pl.kernel,
    out_type=[
        jax.ShapeDtypeStruct((NP, D), jnp.float32),
        jax.ShapeDtypeStruct((N,), jnp.float32),
    ],
    mesh=_MESH,
    scratch_types=[
        pltpu.VMEM((GB, CHUNK), jnp.int32),
        pltpu.VMEM((128,), jnp.float32),
        pltpu.VMEM((1024,), jnp.float32),
        pltpu.VMEM((QCH,), jnp.float32),
        pltpu.VMEM((RCH, D), jnp.float32),
        pltpu.VMEM((RCH,), jnp.float32),
        pltpu.VMEM_SHARED((NP,), jnp.float32),
        pltpu.SemaphoreType.DMA,
    ],
)
def _deg_kernel(idx_hbm, feat_hbm, fsrc_hbm, nr_hbm, idx_v, ones_v, zero_v,
                q_v, frow_v, nrow_v, hist, sem):
    cid = lax.axis_index("c")
    sid = lax.axis_index("s")

    for i in range(8):
        ones_v[pl.ds(i * 16, 16)] = jnp.ones((16,), jnp.float32)
    for i in range(64):
        zero_v[pl.ds(i * 16, 16)] = jnp.zeros((16,), jnp.float32)

    @pl.when(sid < 10)
    def _():
        pltpu.sync_copy(zero_v.at[pl.ds(0, 1000)],
                        hist.at[pl.ds(sid * 1000, 1000)])

    plsc.subcore_barrier()

    def blk(b, carry):
        wid = sid * 2 + b

        def group(g, carry2):
            pltpu.sync_copy(idx_hbm.at[cid, wid, pl.ds(g * GB, GB)], idx_v)

            def body(j, carry3):
                pltpu.sync_copy(ones_v, hist.at[idx_v.at[j]], add=True)
                return carry3

            lax.fori_loop(0, GB, body, 0)
            return carry2

        lax.fori_loop(0, NG, group, 0)
        return carry

    lax.fori_loop(0, 2, blk, 0)

    plsc.subcore_barrier()

    def rs(m, carry):
        k = sid + m * NS

        @pl.when(k < NQ)
        def _():
            pltpu.sync_copy(hist.at[pl.ds(k * QCH, QCH)], q_v)
            for t in range(QCH // 16):
                x = jnp.maximum(q_v[pl.ds(t * 16, 16)], 1.0)
                q_v[pl.ds(t * 16, 16)] = _qrsqrt(x)

            @pl.when(cid == 0)
            def _():
                pltpu.sync_copy(q_v, hist.at[pl.ds(k * QCH, QCH)])

            @pl.when(cid == 1)
            def _():
                pltpu.sync_copy(q_v, nr_hbm.at[pl.ds(k * QCH, QCH)])

        return carry

    lax.fori_loop(0, (NQ + NS - 1) // NS, rs, 0)

    plsc.subcore_barrier()

    @pl.when(cid == 0)
    def _():
        def sc(m, carry):
            k = sid + m * NS

            @pl.when(k < NRCH)
            def _():
                pltpu.sync_copy(feat_hbm.at[pl.ds(k * RCH, RCH)], frow_v)
                pltpu.sync_copy(hist.at[pl.ds(k * RCH, RCH)], nrow_v)

                def rowblk(rb, carry2):
                    nv = nrow_v[pl.ds(rb * 16, 16)]
                    for rr in range(16):
                        r = rb * 16 + rr
                        v = nv[rr]
                        for c in range(D // 16):
                            frow_v[r, pl.ds(c * 16, 16)] = (
                                frow_v[r, pl.ds(c * 16, 16)] * v)
                    return carry2

                lax.fori_loop(0, RCH // 16, rowblk, 0)
                pltpu.sync_copy(frow_v, fsrc_hbm.at[pl.ds(k * RCH, RCH)])

            return carry

        lax.fori_loop(0, (NRCH + NS - 1) // NS, sc, 0)


@functools.partial(
    pl.kernel,
    out_type=jax.ShapeDtypeStruct((NC, N, D), jnp.float32),
    mesh=_MESH,
    scratch_types=[
        pltpu.VMEM((GB, CHUNK), jnp.int32),
        pltpu.VMEM((GB, CHUNK), jnp.int32),
        pltpu.VMEM((CHUNK, D), jnp.float32),
        pltpu.VMEM((CHUNK, D), jnp.float32),
        pltpu.VMEM((RCH, D), jnp.float32),
        pltpu.VMEM((RCH,), jnp.float32),
        pltpu.VMEM_SHARED((NP, D), jnp.float32),
        pltpu.SemaphoreType.DMA,
        pltpu.SemaphoreType.DMA,
    ],
)
def _spmm_kernel(idx_hbm, feat_hbm, nr_hbm, out_hbm, src_v, dst_v, rows_a,
                 rows_b, zrow_v, nrow_v, accum, sem_a, sem_b):
    cid = lax.axis_index("c")
    sid = lax.axis_index("s")
    wid = cid * NS + sid

    def zfill(j, carry):
        for c in range(D // 16):
            zrow_v[j, pl.ds(c * 16, 16)] = jnp.zeros((16,), jnp.float32)
        return carry

    lax.fori_loop(0, RCH, zfill, 0)

    def zacc(m, carry):
        k = sid + m * NS

        @pl.when(k < NRCH)
        def _():
            pltpu.sync_copy(zrow_v, accum.at[pl.ds(k * RCH, RCH)])

        return carry

    lax.fori_loop(0, (NRCH + NS - 1) // NS, zacc, 0)
    plsc.subcore_barrier()

    def group(g, carry):
        pltpu.sync_copy(idx_hbm.at[0, wid, pl.ds(g * GB, GB)], src_v)
        pltpu.sync_copy(idx_hbm.at[1, wid, pl.ds(g * GB, GB)], dst_v)

        pltpu.async_copy(feat_hbm.at[src_v.at[0]], rows_a, sem_a)

        def pair(p, carry2):
            j0 = 2 * p
            pltpu.async_copy(feat_hbm.at[src_v.at[j0 + 1]], rows_b, sem_b)
            pltpu.make_async_copy(feat_hbm.at[src_v.at[j0]], rows_a,
                                  sem_a).wait()
            pltpu.sync_copy(rows_a, accum.at[dst_v.at[j0]], add=True)

            @pl.when(j0 + 2 < GB)
            def _():
                pltpu.async_copy(feat_hbm.at[src_v.at[j0 + 2]], rows_a, sem_a)

            pltpu.make_async_copy(feat_hbm.at[src_v.at[j0 + 1]], rows_b,
                                  sem_b).wait()
            pltpu.sync_copy(rows_b, accum.at[dst_v.at[j0 + 1]], add=True)
            return carry2

        lax.fori_loop(0, GB // 2, pair, 0)
        return carry

    lax.fori_loop(0, NG, group, 0)

    plsc.subcore_barrier()

    def copyout(m, carry):
        k = sid + m * NS

        @pl.when(k < NRCH)
        def _():
            pltpu.sync_copy(accum.at[pl.ds(k * RCH, RCH)], zrow_v)
            pltpu.sync_copy(nr_hbm.at[pl.ds(k * RCH, RCH)], nrow_v)

            def rowblk(rb, carry2):
                nv = nrow_v[pl.ds(rb * 16, 16)]
                for rr in range(16):
                    r = rb * 16 + rr
                    v = nv[rr]
                    for c in range(D // 16):
                        zrow_v[r, pl.ds(c * 16, 16)] = (
                            zrow_v[r, pl.ds(c * 16, 16)] * v)
                return carry2

            lax.fori_loop(0, RCH // 16, rowblk, 0)
            pltpu.sync_copy(zrow_v, out_hbm.at[cid, pl.ds(k * RCH, RCH)])

        return carry

    lax.fori_loop(0, (NRCH + NS - 1) // NS, copyout, 0)


RB = 1000


def _out_body(accp_ref, w_ref, out_ref):
    acc = accp_ref[0] + accp_ref[1]
    out_ref[...] = jnp.dot(acc, w_ref[...],
                           preferred_element_type=jnp.float32)


_combine = pl.pallas_call(
    _out_body,
    grid=(N // RB,),
    in_specs=[
        pl.BlockSpec((NC, RB, D), lambda i: (0, i, 0)),
        pl.BlockSpec((D, D), lambda i: (0, 0)),
    ],
    out_specs=pl.BlockSpec((RB, D), lambda i: (i, 0)),
    out_shape=jax.ShapeDtypeStruct((N, D), jnp.float32),
)


@jax.jit
def kernel(feat, edge_index, weight):
    ei = edge_index.astype(jnp.int32).reshape(2, NW, E // NW)
    tail = jnp.broadcast_to(N + jnp.arange(EPW - E // NW, dtype=jnp.int32),
                            (2, NW, EPW - E // NW))
    idx = jnp.concatenate([ei, tail], axis=2).reshape(2, NW, CPW, CHUNK)
    feat_src, norm_r = _deg_kernel(idx, feat)
    accp = _spmm_kernel(idx, feat_src, norm_r)
    return _combine(accp, weight)


# speedup vs baseline: 2.5673x; 1.0615x over previous
import functools

import jax
import jax.numpy as jnp
from jax import lax
from jax.experimental import pallas as pl
from jax.experimental.pallas import tpu as pltpu
from jax.experimental.pallas import tpu_sc as plsc

N = 10000
NP = 10240
D = 128
E = 320000
NC = 2
NS = 16
NW = NC * NS
CHUNK = 128
EPW = 10240
CPW = EPW // CHUNK
GB = 16
NG = CPW // GB
RCH = 80
SCR = 640
FULL = 8
QCH = 400
NQ = N // QCH
NRCH = N // RCH

_MESH = plsc.VectorSubcoreMesh(core_axis_name="c", subcore_axis_name="s")


def _qrsqrt(x):
    magic = jnp.full((16,), 0x5F3759DF, jnp.int32)
    one = jnp.full((16,), 1, jnp.int32)
    half = jnp.full((16,), 0.5, jnp.float32)
    th = jnp.full((16,), 1.5, jnp.float32)
    i = lax.bitcast_convert_type(x, jnp.int32)
    y = lax.bitcast_convert_type(
        magic - lax.shift_right_arithmetic(i, one), jnp.float32)
    hx = half * x
    for _ in range(3):
        y = y * (th - hx * y * y)
    return y


def _scale_rows(rows_ref, norm_ref, noff):
    def rowblk(rb, carry):
        nv = norm_ref[pl.ds(noff + rb * 16, 16)]
        for rr in range(16):
            r = rb * 16 + rr
            v = nv[rr]
            for c in range(D // 16):
                rows_ref[r, pl.ds(c * 16, 16)] = (
                    rows_ref[r, pl.ds(c * 16, 16)] * v)
        return carry

    lax.fori_loop(0, RCH // 16, rowblk, 0)


@functools.partial(
    pl.kernel,
    out_type=[
        jax.ShapeDtypeStruct((NP, D), jnp.float32),
        jax.ShapeDtypeStruct((NP,), jnp.float32),
    ],
    mesh=_MESH,
    scratch_types=[
        pltpu.VMEM((GB, CHUNK), jnp.int32),
        pltpu.VMEM((128,), jnp.float32),
        pltpu.VMEM((1024,), jnp.float32),
        pltpu.VMEM((QCH,), jnp.float32),
        pltpu.VMEM((RCH, D), jnp.float32),
        pltpu.VMEM((RCH, D), jnp.float32),
        pltpu.VMEM((SCR,), jnp.float32),
        pltpu.VMEM_SHARED((NP,), jnp.float32),
        pltpu.SemaphoreType.DMA,
        pltpu.SemaphoreType.DMA,
    ],
)
def _deg_kernel(idx_hbm, feat_hbm, fsrc_hbm, nr_hbm, idx_v, ones_v, zero_v,
                q_v, frow_a, frow_b, ntile_v, hist, sem_a, sem_b):
    cid = lax.axis_index("c")
    sid = lax.axis_index("s")

    for i in range(8):
        ones_v[pl.ds(i * 16, 16)] = jnp.ones((16,), jnp.float32)
    for i in range(64):
        zero_v[pl.ds(i * 16, 16)] = jnp.zeros((16,), jnp.float32)

    @pl.when(sid < 10)
    def _():
        pltpu.sync_copy(zero_v.at[pl.ds(0, 1000)],
                        hist.at[pl.ds(sid * 1000, 1000)])

    plsc.subcore_barrier()

    def blk(b, carry):
        wid = sid * 2 + b

        def group(g, carry2):
            pltpu.sync_copy(idx_hbm.at[cid, wid, pl.ds(g * GB, GB)], idx_v)

            def body(j, carry3):
                pltpu.sync_copy(ones_v, hist.at[idx_v.at[j]], add=True)
                return carry3

            lax.fori_loop(0, GB, body, 0)
            return carry2

        lax.fori_loop(0, NG, group, 0)
        return carry

    lax.fori_loop(0, 2, blk, 0)

    plsc.subcore_barrier()

    def rs(m, carry):
        k = sid + m * NS

        @pl.when(k < NQ)
        def _():
            pltpu.sync_copy(hist.at[pl.ds(k * QCH, QCH)], q_v)
            for t in range(QCH // 16):
                x = jnp.maximum(q_v[pl.ds(t * 16, 16)], 1.0)
                q_v[pl.ds(t * 16, 16)] = _qrsqrt(x)

            @pl.when(cid == 0)
            def _():
                pltpu.sync_copy(q_v, hist.at[pl.ds(k * QCH, QCH)])

            @pl.when(cid == 1)
            def _():
                pltpu.sync_copy(q_v, nr_hbm.at[pl.ds(k * QCH, QCH)])

        return carry

    lax.fori_loop(0, (NQ + NS - 1) // NS, rs, 0)

    plsc.subcore_barrier()

    @pl.when(cid == 0)
    def _():
        base = sid * SCR
        cnt = jnp.where(sid == NS - 1, 5, FULL)
        pltpu.sync_copy(hist.at[pl.ds(base, SCR)], ntile_v)
        pltpu.async_copy(feat_hbm.at[pl.ds(base, RCH)], frow_a, sem_a)

        def pairblk(q, carry):
            m0 = 2 * q

            @pl.when(m0 + 1 < cnt)
            def _():
                pltpu.async_copy(
                    feat_hbm.at[pl.ds(base + (m0 + 1) * RCH, RCH)],
                    frow_b, sem_b)

            @pl.when(m0 < cnt)
            def _():
                pltpu.make_async_copy(
                    feat_hbm.at[pl.ds(base + m0 * RCH, RCH)], frow_a,
                    sem_a).wait()
                _scale_rows(frow_a, ntile_v, m0 * RCH)
                pltpu.sync_copy(frow_a,
                                fsrc_hbm.at[pl.ds(base + m0 * RCH, RCH)])

            @pl.when(m0 + 2 < cnt)
            def _():
                pltpu.async_copy(
                    feat_hbm.at[pl.ds(base + (m0 + 2) * RCH, RCH)],
                    frow_a, sem_a)

            @pl.when(m0 + 1 < cnt)
            def _():
                pltpu.make_async_copy(
                    feat_hbm.at[pl.ds(base + (m0 + 1) * RCH, RCH)], frow_b,
                    sem_b).wait()
                _scale_rows(frow_b, ntile_v, (m0 + 1) * RCH)
                pltpu.sync_copy(
                    frow_b, fsrc_hbm.at[pl.ds(base + (m0 + 1) * RCH, RCH)])

            return carry

        lax.fori_loop(0, FULL // 2, pairblk, 0)


@functools.partial(
    pl.kernel,
    out_type=jax.ShapeDtypeStruct((NC, N, D), jnp.float32),
    mesh=_MESH,
    scratch_types=[
        pltpu.VMEM((GB, CHUNK), jnp.int32),
        pltpu.VMEM((GB, CHUNK), jnp.int32),
        pltpu.VMEM((CHUNK, D), jnp.float32),
        pltpu.VMEM((CHUNK, D), jnp.float32),
        pltpu.VMEM((RCH, D), jnp.float32),
        pltpu.VMEM((SCR,), jnp.float32),
        pltpu.VMEM_SHARED((NP, D), jnp.float32),
        pltpu.SemaphoreType.DMA,
        pltpu.SemaphoreType.DMA,
    ],
)
def _spmm_kernel(idx_hbm, feat_hbm, nr_hbm, out_hbm, src_v, dst_v, rows_a,
                 rows_b, zrow_v, ntile_v, accum, sem_a, sem_b):
    cid = lax.axis_index("c")
    sid = lax.axis_index("s")
    wid = cid * NS + sid

    def zfill(j, carry):
        for c in range(D // 16):
            zrow_v[j, pl.ds(c * 16, 16)] = jnp.zeros((16,), jnp.float32)
        return carry

    lax.fori_loop(0, RCH, zfill, 0)

    def zacc(m, carry):
        k = sid + m * NS

        @pl.when(k < NRCH)
        def _():
            pltpu.sync_copy(zrow_v, accum.at[pl.ds(k * RCH, RCH)])

        return carry

    lax.fori_loop(0, (NRCH + NS - 1) // NS, zacc, 0)
    plsc.subcore_barrier()

    def group(g, carry):
        pltpu.sync_copy(idx_hbm.at[0, wid, pl.ds(g * GB, GB)], src_v)
        pltpu.sync_copy(idx_hbm.at[1, wid, pl.ds(g * GB, GB)], dst_v)

        pltpu.async_copy(feat_hbm.at[src_v.at[0]], rows_a, sem_a)

        def pair(p, carry2):
            j0 = 2 * p
            pltpu.async_copy(feat_hbm.at[src_v.at[j0 + 1]], rows_b, sem_b)
            pltpu.make_async_copy(feat_hbm.at[src_v.at[j0]], rows_a,
                                  sem_a).wait()
            pltpu.sync_copy(rows_a, accum.at[dst_v.at[j0]], add=True)

            @pl.when(j0 + 2 < GB)
            def _():
                pltpu.async_copy(feat_hbm.at[src_v.at[j0 + 2]], rows_a, sem_a)

            pltpu.make_async_copy(feat_hbm.at[src_v.at[j0 + 1]], rows_b,
                                  sem_b).wait()
            pltpu.sync_copy(rows_b, accum.at[dst_v.at[j0 + 1]], add=True)
            return carry2

        lax.fori_loop(0, GB // 2, pair, 0)
        return carry

    lax.fori_loop(0, NG, group, 0)

    plsc.subcore_barrier()

    base = sid * SCR
    cnt = jnp.where(sid == NS - 1, 5, FULL)
    pltpu.sync_copy(nr_hbm.at[pl.ds(base, SCR)], ntile_v)
    pltpu.async_copy(accum.at[pl.ds(base, RCH)], rows_a.at[pl.ds(0, RCH)],
                     sem_a)

    def co(q, carry):
        m0 = 2 * q

        @pl.when(m0 + 1 < cnt)
        def _():
            pltpu.async_copy(accum.at[pl.ds(base + (m0 + 1) * RCH, RCH)],
                             rows_b.at[pl.ds(0, RCH)], sem_b)

        @pl.when(m0 < cnt)
        def _():
            pltpu.make_async_copy(accum.at[pl.ds(base + m0 * RCH, RCH)],
                                  rows_a.at[pl.ds(0, RCH)], sem_a).wait()
            _scale_rows(rows_a, ntile_v, m0 * RCH)
            pltpu.sync_copy(rows_a.at[pl.ds(0, RCH)],
                            out_hbm.at[cid, pl.ds(base + m0 * RCH, RCH)])

        @pl.when(m0 + 2 < cnt)
        def _():
            pltpu.async_copy(accum.at[pl.ds(base + (m0 + 2) * RCH, RCH)],
                             rows_a.at[pl.ds(0, RCH)], sem_a)

        @pl.when(m0 + 1 < cnt)
        def _():
            pltpu.make_async_copy(accum.at[pl.ds(base + (m0 + 1) * RCH, RCH)],
                                  rows_b.at[pl.ds(0, RCH)], sem_b).wait()
            _scale_rows(rows_b, ntile_v, (m0 + 1) * RCH)
            pltpu.sync_copy(rows_b.at[pl.ds(0, RCH)],
                            out_hbm.at[cid, pl.ds(base + (m0 + 1) * RCH, RCH)])

        return carry

    lax.fori_loop(0, FULL // 2, co, 0)


RB = 1000


def _out_body(accp_ref, w_ref, out_ref):
    acc = accp_ref[0] + accp_ref[1]
    out_ref[...] = jnp.dot(acc, w_ref[...],
                           preferred_element_type=jnp.float32)


_combine = pl.pallas_call(
    _out_body,
    grid=(N // RB,),
    in_specs=[
        pl.BlockSpec((NC, RB, D), lambda i: (0, i, 0)),
        pl.BlockSpec((D, D), lambda i: (0, 0)),
    ],
    out_specs=pl.BlockSpec((RB, D), lambda i: (i, 0)),
    out_shape=jax.ShapeDtypeStruct((N, D), jnp.float32),
)


@jax.jit
def kernel(feat, edge_index, weight):
    ei = edge_index.astype(jnp.int32).reshape(2, NW, E // NW)
    tail = jnp.broadcast_to(N + jnp.arange(EPW - E // NW, dtype=jnp.int32),
                            (2, NW, EPW - E // NW))
    idx = jnp.concatenate([ei, tail], axis=2).reshape(2, NW, CPW, CHUNK)
    feat_src, norm_r = _deg_kernel(idx, feat)
    accp = _spmm_kernel(idx, feat_src, norm_r)
    return _combine(accp, weight)
